# baseline (speedup 1.0000x reference)
import functools

import jax
import jax.numpy as jnp
from jax import lax
from jax.experimental import pallas as pl
from jax.experimental.pallas import tpu as pltpu
from jax.experimental.pallas import tpu_sc as plsc

_NC = 2
_NS = 16
_NW = _NC * _NS
_R = 16
_SR = 64


def _sc_body(row, bpw, cls_hbm, emb_hbm, out_hbm, buf, shared, sem):
    cid = lax.axis_index("c")
    sid = lax.axis_index("s")
    d = 64

    @pl.when(sid == 0)
    def _fill():
        for r in range(_R):
            pltpu.sync_copy(cls_hbm, buf.at[pl.ds(r * row, d)])
            pltpu.sync_copy(emb_hbm, buf.at[pl.ds(r * row + d, row - d)])
        for g in range(_SR // _R):
            pltpu.sync_copy(buf, shared.at[pl.ds(g * _R * row, _R * row)])

    plsc.subcore_barrier()

    wid = sid * _NC + cid
    base = wid * bpw * row
    chunk = _SR * row
    copies = [
        pltpu.make_async_copy(shared, out_hbm.at[pl.ds(base + c * chunk, chunk)], sem)
        for c in range(bpw // _SR)
    ]
    for cp in copies:
        cp.start()
    for cp in copies:
        cp.wait()


def kernel(x, feature_embeddings, cls_token):
    batch = x.shape[0]
    num_feats, d = feature_embeddings.shape
    seq = num_feats + 1
    row = seq * d
    bpw = batch // _NW

    mesh = plsc.VectorSubcoreMesh(core_axis_name="c", subcore_axis_name="s")
    sc_fill = pl.kernel(
        functools.partial(_sc_body, row, bpw),
        out_type=jax.ShapeDtypeStruct((batch * row,), jnp.float32),
        mesh=mesh,
        scratch_types=[
            pltpu.VMEM((_R * row,), jnp.float32),
            pltpu.VMEM_SHARED((_SR * row,), jnp.float32),
            pltpu.SemaphoreType.DMA,
        ],
    )
    out_flat = sc_fill(cls_token.reshape(d), feature_embeddings.reshape(num_feats * d))
    return out_flat.reshape(batch, seq, d)


# speedup vs baseline: 2.4728x; 2.4728x over previous
import jax
import jax.numpy as jnp
from jax.experimental import pallas as pl
from jax.experimental.pallas import tpu as pltpu

_R = 128
_NQ = 8


def _bcast_body(nchunk, comb_ref, out_hbm, buf, sems):
    i = pl.program_id(0)

    @pl.when(i == 0)
    def _fill():
        buf[...] = jnp.broadcast_to(comb_ref[...], buf.shape)

    def _copy(j):
        return pltpu.make_async_copy(
            buf, out_hbm.at[pl.ds(j * _R, _R), :], sems.at[j % _NQ]
        )

    @pl.when(i >= _NQ)
    def _retire():
        _copy(i - _NQ).wait()

    _copy(i).start()

    @pl.when(i == nchunk - 1)
    def _drain():
        for k in range(_NQ):
            _copy(i - _NQ + 1 + k).wait()


def kernel(x, feature_embeddings, cls_token):
    batch = x.shape[0]
    num_feats, d = feature_embeddings.shape
    seq = num_feats + 1
    row = seq * d
    nchunk = batch // _R
    comb = jnp.concatenate([cls_token[0], feature_embeddings], axis=0)
    comb_flat = comb.reshape(1, row)
    out2d = pl.pallas_call(
        lambda *refs: _bcast_body(nchunk, *refs),
        grid=(nchunk,),
        in_specs=[pl.BlockSpec((1, row), lambda i: (0, 0))],
        out_specs=pl.BlockSpec(memory_space=pl.ANY),
        out_shape=jax.ShapeDtypeStruct((batch, row), jnp.float32),
        scratch_shapes=[
            pltpu.VMEM((_R, row), jnp.float32),
            pltpu.SemaphoreType.DMA((_NQ,)),
        ],
    )(comb_flat)
    return out2d.reshape(batch, seq, d)
